# Initial kernel scaffold; baseline (speedup 1.0000x reference)
import jax
import jax.numpy as jnp
from jax.experimental import pallas as pl


def kernel(x, src_tgt, emb_x, emb_y, emb_z, src_pos_x, src_pos_y, src_pos_z):
    raise NotImplementedError("write your pallas kernel here")


# speedup vs baseline: 3.6836x; 3.6836x over previous
import jax
import jax.numpy as jnp
from jax import lax
from jax.experimental import pallas as pl

BS = 512


def _body(idx_ref, e_ref, x_ref, out_ref):
    idx = idx_ref[0]
    iot = lax.broadcasted_iota(jnp.int32, (BS, 72), 1)
    oh = (
        (iot == idx[0][:, None]).astype(jnp.float32)
        + (iot == idx[1][:, None]).astype(jnp.float32)
        + (iot == idx[2][:, None]).astype(jnp.float32)
    )
    pos = jnp.dot(oh, e_ref[...], preferred_element_type=jnp.float32)
    out_ref[...] = x_ref[...] + pos[None, :, :]


def kernel(x, src_tgt, emb_x, emb_y, emb_z, src_pos_x, src_pos_y, src_pos_z):
    B, S, D = x.shape
    d3 = emb_x.shape[1]
    nx, ny, nz = emb_x.shape[0], emb_y.shape[0], emb_z.shape[0]

    is_src = (src_tgt != 0)
    sx = jnp.concatenate([jnp.array([nx - 1], jnp.int32), src_pos_x])[:S]
    sy = jnp.concatenate([jnp.array([ny - 1], jnp.int32), src_pos_y])[:S]
    sz = jnp.concatenate([jnp.array([nz - 1], jnp.int32), src_pos_z])[:S]
    px = jnp.where(is_src, src_pos_x, sx)
    py = jnp.where(is_src, src_pos_y, sy) + nx
    pz = jnp.where(is_src, src_pos_z, sz) + nx + ny

    rows = nx + ny + nz
    rows_pad = 72
    E = jnp.zeros((rows_pad, D), jnp.float32)
    E = E.at[:nx, :d3].set(emb_x)
    E = E.at[nx:nx + ny, d3:2 * d3].set(emb_y)
    E = E.at[nx + ny:rows, 2 * d3:].set(emb_z)

    nb = S // BS
    idx = jnp.stack([px, py, pz]).reshape(3, nb, BS).transpose(1, 0, 2)

    out = pl.pallas_call(
        _body,
        grid=(nb,),
        in_specs=[
            pl.BlockSpec((1, 3, BS), lambda i: (i, 0, 0)),
            pl.BlockSpec((rows_pad, D), lambda i: (0, 0)),
            pl.BlockSpec((B, BS, D), lambda i: (0, i, 0)),
        ],
        out_specs=pl.BlockSpec((B, BS, D), lambda i: (0, i, 0)),
        out_shape=jax.ShapeDtypeStruct((B, S, D), jnp.float32),
    )(idx, E, x)
    return out
